# baseline (speedup 1.0000x reference)
import jax
import jax.numpy as jnp
from jax import lax
from jax.experimental import pallas as pl
from jax.experimental.pallas import tpu as pltpu
from jax.experimental.pallas import tpu_sc as plsc

B, C, H, W = 8, 3, 512, 512
NC, NS, L = 2, 16, 16
IMGS_PER_CORE = B // NC
WPI = NS // IMGS_PER_CORE
ROWS_PER_WORKER = H // WPI
RB = 16
NCHUNK = ROWS_PER_WORKER // RB
NJ = W // L
NACC = 8


def _sc_body(seg_hbm, out_hbm, buf, stage, gather, feat, shared, sem0, sem1):
    c = lax.axis_index("c")
    s = lax.axis_index("s")
    img = s // WPI
    q = s % WPI
    b = c * IMGS_PER_CORE + img
    r0 = q * ROWS_PER_WORKER

    sems = (sem0, sem1)

    def start(g):
        return pltpu.async_copy(
            seg_hbm.at[b, :, pl.ds(r0 + g * RB, RB), :],
            buf.at[g % 2],
            sems[g % 2],
        )

    iota_f = lax.iota(jnp.int32, L).astype(jnp.float32)
    zv = jnp.zeros((L,), jnp.float32)
    onev = jnp.full((L,), 1.0, jnp.float32)
    bigv = jnp.full((L,), float(H + W), jnp.float32)
    negv = jnp.full((L,), -1.0, jnp.float32)

    zu = jnp.zeros((L,), jnp.uint32)
    oneu = jnp.full((L,), 1, jnp.uint32)
    zi = jnp.zeros((L,), jnp.int32)

    carry = (zi, zi, zv, zv, zu, bigv, negv)

    desc = [None, None]
    desc[0] = start(0)
    for g in range(NCHUNK):
        if g + 1 < NCHUNK:
            desc[(g + 1) % 2] = start(g + 1)
        desc[g % 2].wait()

        def row_body(r, car, g=g):
            cnt_p, cnt_v, pw, vw, colbits, rmin, rmax = car
            prev_p = cnt_p

            def grp_body(j, gcar):
                cnt_p, cnt_v, pw, vw, colbits = gcar
                l0 = buf[g % 2, 0, r, pl.ds(j * L, L)]
                l1 = buf[g % 2, 1, r, pl.ds(j * L, L)]
                l2 = buf[g % 2, 2, r, pl.ds(j * L, L)]
                d1 = l1 - l0
                d2 = l2 - l0
                f1 = jnp.exp(d1)
                f2 = jnp.exp(d2)
                inv = 1.0 / (1.0 + f1 + f2)
                c10 = d1 > 0.0
                c20 = d2 > 0.0
                c12 = l1 >= l2
                plq = c10 & c12
                ves = c20 & (~c12)
                cnt_p = cnt_p + plsc.all_reduce_population_count(plq)
                cnt_v = cnt_v + plsc.all_reduce_population_count(ves)
                pw = pw + jnp.where(plq, f1 * inv, zv)
                vw = vw + jnp.where(ves, f2 * inv, zv)
                colbits = colbits | jnp.where(plq, oneu << j, zu)
                return (cnt_p, cnt_v, pw, vw, colbits)

            cnt_p, cnt_v, pw, vw, colbits = plsc.parallel_loop(
                0, NJ, 1, unroll=8, carry=(cnt_p, cnt_v, pw, vw, colbits)
            )(grp_body)
            rowf = (r0 + g * RB + r).astype(jnp.float32)
            rmask = cnt_p > prev_p
            rmin = jnp.minimum(rmin, jnp.where(rmask, rowf, bigv))
            rmax = jnp.maximum(rmax, jnp.where(rmask, rowf, negv))
            return (cnt_p, cnt_v, pw, vw, colbits, rmin, rmax)

        carry = lax.fori_loop(0, RB, row_body, carry)

    cnt_p, cnt_v, pw, vw, colbits, rmin, rmax = carry

    def col_fold(j, cc):
        cmin, cmax = cc
        anyv = ((colbits >> j) & oneu) > 0
        colf = iota_f + (j * L).astype(jnp.float32)
        cmin = jnp.minimum(cmin, jnp.where(anyv, colf, bigv))
        cmax = jnp.maximum(cmax, jnp.where(anyv, colf, negv))
        return (cmin, cmax)

    cmin, cmax = lax.fori_loop(0, NJ, col_fold, (bigv, negv))
    carry = (cnt_p.astype(jnp.float32), cnt_v.astype(jnp.float32),
             pw, vw, cmin, cmax, rmin, rmax)

    for i in range(NACC):
        stage[i, :] = carry[i]
    pltpu.sync_copy(stage, shared.at[s])
    plsc.subcore_barrier()

    @pl.when(q == 0)
    def _combine():
        pltpu.sync_copy(shared.at[pl.ds(s, WPI)], gather)
        cnt_p = gather[0, 0, :] + gather[1, 0, :] + gather[2, 0, :] + gather[3, 0, :]
        cnt_v = gather[0, 1, :] + gather[1, 1, :] + gather[2, 1, :] + gather[3, 1, :]
        pw = gather[0, 2, :] + gather[1, 2, :] + gather[2, 2, :] + gather[3, 2, :]
        vw = gather[0, 3, :] + gather[1, 3, :] + gather[2, 3, :] + gather[3, 3, :]
        cmin = jnp.minimum(jnp.minimum(gather[0, 4, :], gather[1, 4, :]),
                           jnp.minimum(gather[2, 4, :], gather[3, 4, :]))
        cmax = jnp.maximum(jnp.maximum(gather[0, 5, :], gather[1, 5, :]),
                           jnp.maximum(gather[2, 5, :], gather[3, 5, :]))
        rmin = jnp.minimum(jnp.minimum(gather[0, 6, :], gather[1, 6, :]),
                           jnp.minimum(gather[2, 6, :], gather[3, 6, :]))
        rmax = jnp.maximum(jnp.maximum(gather[0, 7, :], gather[1, 7, :]),
                           jnp.maximum(gather[2, 7, :], gather[3, 7, :]))

        sp = jnp.max(cnt_p)
        sv = jnp.max(cnt_v)
        spw = jnp.sum(pw)
        svw = jnp.sum(vw)
        cmn = jnp.min(cmin)
        cmx = jnp.max(cmax)
        rmn = jnp.min(rmin)
        rmx = jnp.max(rmax)

        fg = sp + sv
        hp = sp > 0.0
        zero = jnp.float32(0.0)
        hr = jnp.where(hp, rmx - rmn, zero)
        wr = jnp.where(hp, cmx - cmn, zero)

        nums = (
            sp,
            sp,
            spw,
            hr,
            wr,
            2.0 * (hr + wr),
            svw,
            jnp.where(hp, spw, zero),
            fg,
            sp,
        )
        dens = (
            sv + 1e-6,
            fg + 1e-6,
            1.0,
            float(H),
            float(W),
            float(H + W),
            1.0,
            sp + 1e-6,
            float(H * W),
            float(H * W),
        )
        lane = lax.iota(jnp.int32, L)
        num = zv
        den = onev
        for k in range(10):
            sel = lane == k
            num = jnp.where(sel, nums[k], num)
            den = jnp.where(sel, dens[k], den)
        feat[:] = num / den
        pltpu.sync_copy(feat, out_hbm.at[b])


_mesh = plsc.VectorSubcoreMesh(
    core_axis_name="c", subcore_axis_name="s", num_cores=NC, num_subcores=NS
)

_sc_kernel = pl.kernel(
    _sc_body,
    out_type=jax.ShapeDtypeStruct((B, L), jnp.float32),
    mesh=_mesh,
    scratch_types=[
        pltpu.VMEM((2, C, RB, W), jnp.float32),
        pltpu.VMEM((NACC, L), jnp.float32),
        pltpu.VMEM((WPI, NACC, L), jnp.float32),
        pltpu.VMEM((L,), jnp.float32),
        pltpu.VMEM_SHARED((NS, NACC, L), jnp.float32),
        pltpu.SemaphoreType.DMA,
        pltpu.SemaphoreType.DMA,
    ],
    compiler_params=pltpu.CompilerParams(needs_layout_passes=False),
)


@jax.jit
def kernel(seg_logits):
    out = _sc_kernel(seg_logits)
    return out[:, :10]


# speedup vs baseline: 1.1759x; 1.1759x over previous
import jax
import jax.numpy as jnp
from jax import lax
from jax.experimental import pallas as pl
from jax.experimental.pallas import tpu as pltpu
from jax.experimental.pallas import tpu_sc as plsc

B, C, H, W = 8, 3, 512, 512
NC, NS, L = 2, 16, 16
IMGS_PER_CORE = B // NC
WPI = NS // IMGS_PER_CORE
ROWS_PER_WORKER = H // WPI
RB = 16
NCHUNK = ROWS_PER_WORKER // RB
NJ = W // L
NACC = 8


def _sc_body(seg_hbm, out_hbm, buf, stage, gather, feat, shared, sem0, sem1):
    c = lax.axis_index("c")
    s = lax.axis_index("s")
    img = s // WPI
    q = s % WPI
    b = c * IMGS_PER_CORE + img
    r0 = q * ROWS_PER_WORKER

    sems = (sem0, sem1)

    def start(g):
        return pltpu.async_copy(
            seg_hbm.at[b, :, pl.ds(r0 + g * RB, RB), :],
            buf.at[g % 2],
            sems[g % 2],
        )

    iota_f = lax.iota(jnp.int32, L).astype(jnp.float32)
    zv = jnp.zeros((L,), jnp.float32)
    onev = jnp.full((L,), 1.0, jnp.float32)
    bigv = jnp.full((L,), float(H + W), jnp.float32)
    negv = jnp.full((L,), -1.0, jnp.float32)

    zu = jnp.zeros((L,), jnp.uint32)
    oneu = jnp.full((L,), 1, jnp.uint32)
    zi = jnp.zeros((L,), jnp.int32)

    carry = (zi, zi, zi, zv, zv, zu, bigv, negv)

    U = 2

    desc = [None, None]
    desc[0] = start(0)
    for g in range(NCHUNK):
        if g + 1 < NCHUNK:
            desc[(g + 1) % 2] = start(g + 1)
        desc[g % 2].wait()

        def row_body(rr, car, g=g):
            cnt_p0, cnt_p1, cnt_v, pw, vw, colbits, rmin, rmax = car
            prev0 = cnt_p0
            prev1 = cnt_p1
            ra = 2 * rr
            rb = 2 * rr + 1

            def grp_body(jj, gcar):
                cnt_p0, cnt_p1, cnt_v, pw, vw, colbits = gcar
                for u in range(U):
                    j = jj * U + u
                    for r, which in ((ra, 0), (rb, 1)):
                        l0 = buf[g % 2, 0, r, pl.ds(j * L, L)]
                        l1 = buf[g % 2, 1, r, pl.ds(j * L, L)]
                        l2 = buf[g % 2, 2, r, pl.ds(j * L, L)]
                        d1 = l1 - l0
                        d2 = l2 - l0
                        f1 = jnp.exp(d1)
                        f2 = jnp.exp(d2)
                        inv = 1.0 / (1.0 + f1 + f2)
                        c10 = d1 > 0.0
                        c20 = d2 > 0.0
                        c12 = l1 >= l2
                        plq = c10 & c12
                        ves = c20 & (~c12)
                        pc = plsc.all_reduce_population_count(plq)
                        if which == 0:
                            cnt_p0 = cnt_p0 + pc
                        else:
                            cnt_p1 = cnt_p1 + pc
                        cnt_v = cnt_v + plsc.all_reduce_population_count(ves)
                        pw = pw + jnp.where(plq, f1 * inv, zv)
                        vw = vw + jnp.where(ves, f2 * inv, zv)
                        colbits = colbits | jnp.where(plq, oneu << j, zu)
                return (cnt_p0, cnt_p1, cnt_v, pw, vw, colbits)

            cnt_p0, cnt_p1, cnt_v, pw, vw, colbits = lax.fori_loop(
                0, NJ // U, grp_body, (cnt_p0, cnt_p1, cnt_v, pw, vw, colbits)
            )
            rowfa = (r0 + g * RB + ra).astype(jnp.float32)
            rowfb = (r0 + g * RB + rb).astype(jnp.float32)
            rmask0 = cnt_p0 > prev0
            rmask1 = cnt_p1 > prev1
            rmin = jnp.minimum(rmin, jnp.where(rmask0, rowfa, bigv))
            rmax = jnp.maximum(rmax, jnp.where(rmask0, rowfa, negv))
            rmin = jnp.minimum(rmin, jnp.where(rmask1, rowfb, bigv))
            rmax = jnp.maximum(rmax, jnp.where(rmask1, rowfb, negv))
            return (cnt_p0, cnt_p1, cnt_v, pw, vw, colbits, rmin, rmax)

        carry = lax.fori_loop(0, RB // 2, row_body, carry)

    cnt_p0, cnt_p1, cnt_v, pw, vw, colbits, rmin, rmax = carry
    cnt_p = cnt_p0 + cnt_p1

    def col_fold(j, cc):
        cmin, cmax = cc
        anyv = ((colbits >> j) & oneu) > 0
        colf = iota_f + (j * L).astype(jnp.float32)
        cmin = jnp.minimum(cmin, jnp.where(anyv, colf, bigv))
        cmax = jnp.maximum(cmax, jnp.where(anyv, colf, negv))
        return (cmin, cmax)

    cmin, cmax = lax.fori_loop(0, NJ, col_fold, (bigv, negv))
    carry = (cnt_p.astype(jnp.float32), cnt_v.astype(jnp.float32),
             pw, vw, cmin, cmax, rmin, rmax)

    for i in range(NACC):
        stage[i, :] = carry[i]
    pltpu.sync_copy(stage, shared.at[s])
    plsc.subcore_barrier()

    @pl.when(q == 0)
    def _combine():
        pltpu.sync_copy(shared.at[pl.ds(s, WPI)], gather)
        cnt_p = gather[0, 0, :] + gather[1, 0, :] + gather[2, 0, :] + gather[3, 0, :]
        cnt_v = gather[0, 1, :] + gather[1, 1, :] + gather[2, 1, :] + gather[3, 1, :]
        pw = gather[0, 2, :] + gather[1, 2, :] + gather[2, 2, :] + gather[3, 2, :]
        vw = gather[0, 3, :] + gather[1, 3, :] + gather[2, 3, :] + gather[3, 3, :]
        cmin = jnp.minimum(jnp.minimum(gather[0, 4, :], gather[1, 4, :]),
                           jnp.minimum(gather[2, 4, :], gather[3, 4, :]))
        cmax = jnp.maximum(jnp.maximum(gather[0, 5, :], gather[1, 5, :]),
                           jnp.maximum(gather[2, 5, :], gather[3, 5, :]))
        rmin = jnp.minimum(jnp.minimum(gather[0, 6, :], gather[1, 6, :]),
                           jnp.minimum(gather[2, 6, :], gather[3, 6, :]))
        rmax = jnp.maximum(jnp.maximum(gather[0, 7, :], gather[1, 7, :]),
                           jnp.maximum(gather[2, 7, :], gather[3, 7, :]))

        sp = jnp.max(cnt_p)
        sv = jnp.max(cnt_v)
        spw = jnp.sum(pw)
        svw = jnp.sum(vw)
        cmn = jnp.min(cmin)
        cmx = jnp.max(cmax)
        rmn = jnp.min(rmin)
        rmx = jnp.max(rmax)

        fg = sp + sv
        hp = sp > 0.0
        zero = jnp.float32(0.0)
        hr = jnp.where(hp, rmx - rmn, zero)
        wr = jnp.where(hp, cmx - cmn, zero)

        nums = (
            sp,
            sp,
            spw,
            hr,
            wr,
            2.0 * (hr + wr),
            svw,
            jnp.where(hp, spw, zero),
            fg,
            sp,
        )
        dens = (
            sv + 1e-6,
            fg + 1e-6,
            1.0,
            float(H),
            float(W),
            float(H + W),
            1.0,
            sp + 1e-6,
            float(H * W),
            float(H * W),
        )
        lane = lax.iota(jnp.int32, L)
        num = zv
        den = onev
        for k in range(10):
            sel = lane == k
            num = jnp.where(sel, nums[k], num)
            den = jnp.where(sel, dens[k], den)
        feat[:] = num / den
        pltpu.sync_copy(feat, out_hbm.at[b])


_mesh = plsc.VectorSubcoreMesh(
    core_axis_name="c", subcore_axis_name="s", num_cores=NC, num_subcores=NS
)

_sc_kernel = pl.kernel(
    _sc_body,
    out_type=jax.ShapeDtypeStruct((B, L), jnp.float32),
    mesh=_mesh,
    scratch_types=[
        pltpu.VMEM((2, C, RB, W), jnp.float32),
        pltpu.VMEM((NACC, L), jnp.float32),
        pltpu.VMEM((WPI, NACC, L), jnp.float32),
        pltpu.VMEM((L,), jnp.float32),
        pltpu.VMEM_SHARED((NS, NACC, L), jnp.float32),
        pltpu.SemaphoreType.DMA,
        pltpu.SemaphoreType.DMA,
    ],
    compiler_params=pltpu.CompilerParams(needs_layout_passes=False),
)


@jax.jit
def kernel(seg_logits):
    out = _sc_kernel(seg_logits)
    return out[:, :10]


# speedup vs baseline: 1.2257x; 1.0424x over previous
import jax
import jax.numpy as jnp
from jax import lax
from jax.experimental import pallas as pl
from jax.experimental.pallas import tpu as pltpu
from jax.experimental.pallas import tpu_sc as plsc

B, C, H, W = 8, 3, 512, 512
NC, NS, L = 2, 16, 16
IMGS_PER_CORE = B // NC
WPI = NS // IMGS_PER_CORE
ROWS_PER_WORKER = H // WPI
RB = 16
NCHUNK = ROWS_PER_WORKER // RB
NJ = W // L
NACC = 8


def _sc_body(seg_hbm, out_hbm, buf, stage, gather, feat, shared, sem0, sem1):
    c = lax.axis_index("c")
    s = lax.axis_index("s")
    img = s // WPI
    q = s % WPI
    b = c * IMGS_PER_CORE + img
    r0 = q * ROWS_PER_WORKER

    sems = (sem0, sem1)

    def start(g):
        return pltpu.async_copy(
            seg_hbm.at[b, :, pl.ds(r0 + g * RB, RB), :],
            buf.at[g % 2],
            sems[g % 2],
        )

    iota_f = lax.iota(jnp.int32, L).astype(jnp.float32)
    zv = jnp.zeros((L,), jnp.float32)
    onev = jnp.full((L,), 1.0, jnp.float32)
    bigv = jnp.full((L,), float(H + W), jnp.float32)
    negv = jnp.full((L,), -1.0, jnp.float32)

    zu = jnp.zeros((L,), jnp.uint32)
    oneu = jnp.full((L,), 1, jnp.uint32)
    zi = jnp.zeros((L,), jnp.int32)

    carry = (zi, zi, zi, zv, zv, zu, bigv, negv)

    U = 2

    desc = [None, None]
    desc[0] = start(0)
    for g in range(NCHUNK):
        if g + 1 < NCHUNK:
            desc[(g + 1) % 2] = start(g + 1)
        desc[g % 2].wait()

        def row_body(rr, car, g=g):
            cnt_p0, cnt_p1, cnt_v, pw, vw, colbits, rmin, rmax = car
            prev0 = cnt_p0
            prev1 = cnt_p1
            ra = 2 * rr
            rb = 2 * rr + 1

            def grp_body(jj, gcar):
                cnt_p0, cnt_p1, cnt_v, pw, vw, colbits, bb = gcar
                for u in range(U):
                    j = jj * U + u
                    for r, which in ((ra, 0), (rb, 1)):
                        l0 = buf[g % 2, 0, r, pl.ds(j * L, L)]
                        l1 = buf[g % 2, 1, r, pl.ds(j * L, L)]
                        l2 = buf[g % 2, 2, r, pl.ds(j * L, L)]
                        d1 = l1 - l0
                        d2 = l2 - l0
                        f1 = jnp.exp(d1)
                        f2 = jnp.exp(d2)
                        inv = 1.0 / (1.0 + f1 + f2)
                        fgm = jnp.maximum(l1, l2) > l0
                        c12 = l1 >= l2
                        plq = fgm & c12
                        ves = fgm ^ plq
                        pc = plsc.all_reduce_population_count(plq)
                        if which == 0:
                            cnt_p0 = cnt_p0 + pc
                        else:
                            cnt_p1 = cnt_p1 + pc
                        cnt_v = cnt_v + plsc.all_reduce_population_count(ves)
                        pw = pw + jnp.where(plq, f1 * inv, zv)
                        vw = vw + jnp.where(ves, f2 * inv, zv)
                        colbits = colbits | jnp.where(plq, bb << u, zu)
                bb = bb << U
                return (cnt_p0, cnt_p1, cnt_v, pw, vw, colbits, bb)

            cnt_p0, cnt_p1, cnt_v, pw, vw, colbits, _ = lax.fori_loop(
                0, NJ // U, grp_body,
                (cnt_p0, cnt_p1, cnt_v, pw, vw, colbits, oneu)
            )
            rowfa = (r0 + g * RB + ra).astype(jnp.float32)
            rowfb = (r0 + g * RB + rb).astype(jnp.float32)
            rmask0 = cnt_p0 > prev0
            rmask1 = cnt_p1 > prev1
            rmin = jnp.minimum(rmin, jnp.where(rmask0, rowfa, bigv))
            rmax = jnp.maximum(rmax, jnp.where(rmask0, rowfa, negv))
            rmin = jnp.minimum(rmin, jnp.where(rmask1, rowfb, bigv))
            rmax = jnp.maximum(rmax, jnp.where(rmask1, rowfb, negv))
            return (cnt_p0, cnt_p1, cnt_v, pw, vw, colbits, rmin, rmax)

        carry = lax.fori_loop(0, RB // 2, row_body, carry)

    cnt_p0, cnt_p1, cnt_v, pw, vw, colbits, rmin, rmax = carry
    cnt_p = cnt_p0 + cnt_p1

    def col_fold(j, cc):
        cmin, cmax = cc
        anyv = ((colbits >> j) & oneu) > 0
        colf = iota_f + (j * L).astype(jnp.float32)
        cmin = jnp.minimum(cmin, jnp.where(anyv, colf, bigv))
        cmax = jnp.maximum(cmax, jnp.where(anyv, colf, negv))
        return (cmin, cmax)

    cmin, cmax = lax.fori_loop(0, NJ, col_fold, (bigv, negv))
    carry = (cnt_p.astype(jnp.float32), cnt_v.astype(jnp.float32),
             pw, vw, cmin, cmax, rmin, rmax)

    for i in range(NACC):
        stage[i, :] = carry[i]
    pltpu.sync_copy(stage, shared.at[s])
    plsc.subcore_barrier()

    @pl.when(q == 0)
    def _combine():
        pltpu.sync_copy(shared.at[pl.ds(s, WPI)], gather)
        cnt_p = gather[0, 0, :] + gather[1, 0, :] + gather[2, 0, :] + gather[3, 0, :]
        cnt_v = gather[0, 1, :] + gather[1, 1, :] + gather[2, 1, :] + gather[3, 1, :]
        pw = gather[0, 2, :] + gather[1, 2, :] + gather[2, 2, :] + gather[3, 2, :]
        vw = gather[0, 3, :] + gather[1, 3, :] + gather[2, 3, :] + gather[3, 3, :]
        cmin = jnp.minimum(jnp.minimum(gather[0, 4, :], gather[1, 4, :]),
                           jnp.minimum(gather[2, 4, :], gather[3, 4, :]))
        cmax = jnp.maximum(jnp.maximum(gather[0, 5, :], gather[1, 5, :]),
                           jnp.maximum(gather[2, 5, :], gather[3, 5, :]))
        rmin = jnp.minimum(jnp.minimum(gather[0, 6, :], gather[1, 6, :]),
                           jnp.minimum(gather[2, 6, :], gather[3, 6, :]))
        rmax = jnp.maximum(jnp.maximum(gather[0, 7, :], gather[1, 7, :]),
                           jnp.maximum(gather[2, 7, :], gather[3, 7, :]))

        sp = jnp.max(cnt_p)
        sv = jnp.max(cnt_v)
        spw = jnp.sum(pw)
        svw = jnp.sum(vw)
        cmn = jnp.min(cmin)
        cmx = jnp.max(cmax)
        rmn = jnp.min(rmin)
        rmx = jnp.max(rmax)

        fg = sp + sv
        hp = sp > 0.0
        zero = jnp.float32(0.0)
        hr = jnp.where(hp, rmx - rmn, zero)
        wr = jnp.where(hp, cmx - cmn, zero)

        nums = (
            sp,
            sp,
            spw,
            hr,
            wr,
            2.0 * (hr + wr),
            svw,
            jnp.where(hp, spw, zero),
            fg,
            sp,
        )
        dens = (
            sv + 1e-6,
            fg + 1e-6,
            1.0,
            float(H),
            float(W),
            float(H + W),
            1.0,
            sp + 1e-6,
            float(H * W),
            float(H * W),
        )
        lane = lax.iota(jnp.int32, L)
        num = zv
        den = onev
        for k in range(10):
            sel = lane == k
            num = jnp.where(sel, nums[k], num)
            den = jnp.where(sel, dens[k], den)
        feat[:] = num / den
        pltpu.sync_copy(feat, out_hbm.at[b])


_mesh = plsc.VectorSubcoreMesh(
    core_axis_name="c", subcore_axis_name="s", num_cores=NC, num_subcores=NS
)

_sc_kernel = pl.kernel(
    _sc_body,
    out_type=jax.ShapeDtypeStruct((B, L), jnp.float32),
    mesh=_mesh,
    scratch_types=[
        pltpu.VMEM((2, C, RB, W), jnp.float32),
        pltpu.VMEM((NACC, L), jnp.float32),
        pltpu.VMEM((WPI, NACC, L), jnp.float32),
        pltpu.VMEM((L,), jnp.float32),
        pltpu.VMEM_SHARED((NS, NACC, L), jnp.float32),
        pltpu.SemaphoreType.DMA,
        pltpu.SemaphoreType.DMA,
    ],
    compiler_params=pltpu.CompilerParams(needs_layout_passes=False),
)


@jax.jit
def kernel(seg_logits):
    out = _sc_kernel(seg_logits)
    return out[:, :10]


# speedup vs baseline: 1.2260x; 1.0002x over previous
import jax
import jax.numpy as jnp
from jax import lax
from jax.experimental import pallas as pl
from jax.experimental.pallas import tpu as pltpu
from jax.experimental.pallas import tpu_sc as plsc

B, C, H, W = 8, 3, 512, 512
NC, NS, L = 2, 16, 16
IMGS_PER_CORE = B // NC
WPI = NS // IMGS_PER_CORE
ROWS_PER_WORKER = H // WPI
RB = 16
NCHUNK = ROWS_PER_WORKER // RB
NJ = W // L
NACC = 8


def _sc_body(seg_hbm, out_hbm, buf, stage, gather, feat, shared, sem0, sem1):
    c = lax.axis_index("c")
    s = lax.axis_index("s")
    img = s // WPI
    q = s % WPI
    b = c * IMGS_PER_CORE + img
    r0 = q * ROWS_PER_WORKER

    sems = (sem0, sem1)

    def start(g):
        return pltpu.async_copy(
            seg_hbm.at[b, :, pl.ds(r0 + g * RB, RB), :],
            buf.at[g % 2],
            sems[g % 2],
        )

    iota_f = lax.iota(jnp.int32, L).astype(jnp.float32)
    zv = jnp.zeros((L,), jnp.float32)
    onev = jnp.full((L,), 1.0, jnp.float32)
    bigv = jnp.full((L,), float(H + W), jnp.float32)
    negv = jnp.full((L,), -1.0, jnp.float32)

    zu = jnp.zeros((L,), jnp.uint32)
    oneu = jnp.full((L,), 1, jnp.uint32)
    zi = jnp.zeros((L,), jnp.int32)

    carry = (zi, zi, zi, zv, zv, zu, bigv, negv)

    U = 2

    desc = [None, None]
    desc[0] = start(0)
    for g in range(NCHUNK):
        if g + 1 < NCHUNK:
            desc[(g + 1) % 2] = start(g + 1)
        desc[g % 2].wait()

        def row_body(rr, car, g=g):
            cnt_p0, cnt_p1, cnt_v, pw, vw, colbits, rmin, rmax = car
            prev0 = cnt_p0
            prev1 = cnt_p1
            ra = 2 * rr
            rb = 2 * rr + 1

            def grp_body(jj, gcar):
                cnt_p0, cnt_p1, cnt_v, pw, vw, colbits, bb = gcar
                for u in range(U):
                    j = jj * U + u
                    for r, which in ((ra, 0), (rb, 1)):
                        l0 = buf[g % 2, 0, r, pl.ds(j * L, L)]
                        l1 = buf[g % 2, 1, r, pl.ds(j * L, L)]
                        l2 = buf[g % 2, 2, r, pl.ds(j * L, L)]
                        d1 = l1 - l0
                        d2 = l2 - l0
                        f1 = jnp.exp(d1)
                        f2 = jnp.exp(d2)
                        inv = 1.0 / (1.0 + f1 + f2)
                        fgm = jnp.maximum(l1, l2) > l0
                        c12 = l1 >= l2
                        plq = fgm & c12
                        ves = fgm ^ plq
                        pc = plsc.all_reduce_population_count(plq)
                        if which == 0:
                            cnt_p0 = cnt_p0 + pc
                        else:
                            cnt_p1 = cnt_p1 + pc
                        cnt_v = cnt_v + plsc.all_reduce_population_count(ves)
                        pw = pw + jnp.where(plq, f1 * inv, zv)
                        vw = vw + jnp.where(ves, f2 * inv, zv)
                        colbits = colbits | jnp.where(plq, bb << u, zu)
                bb = bb << U
                return (cnt_p0, cnt_p1, cnt_v, pw, vw, colbits, bb)

            cnt_p0, cnt_p1, cnt_v, pw, vw, colbits, _ = lax.fori_loop(
                0, NJ // U, grp_body,
                (cnt_p0, cnt_p1, cnt_v, pw, vw, colbits, oneu)
            )
            rowfa = (r0 + g * RB + ra).astype(jnp.float32)
            rowfb = (r0 + g * RB + rb).astype(jnp.float32)
            rmask0 = cnt_p0 > prev0
            rmask1 = cnt_p1 > prev1
            rmin = jnp.minimum(rmin, jnp.where(rmask0, rowfa, bigv))
            rmax = jnp.maximum(rmax, jnp.where(rmask0, rowfa, negv))
            rmin = jnp.minimum(rmin, jnp.where(rmask1, rowfb, bigv))
            rmax = jnp.maximum(rmax, jnp.where(rmask1, rowfb, negv))
            return (cnt_p0, cnt_p1, cnt_v, pw, vw, colbits, rmin, rmax)

        carry = lax.fori_loop(0, RB // 2, row_body, carry)

    cnt_p0, cnt_p1, cnt_v, pw, vw, colbits, rmin, rmax = carry
    cnt_p = cnt_p0 + cnt_p1

    def col_fold(j, cc):
        cmin, cmax = cc
        anyv = ((colbits >> j) & oneu) > 0
        colf = iota_f + (j * L).astype(jnp.float32)
        cmin = jnp.minimum(cmin, jnp.where(anyv, colf, bigv))
        cmax = jnp.maximum(cmax, jnp.where(anyv, colf, negv))
        return (cmin, cmax)

    cmin, cmax = lax.fori_loop(0, NJ, col_fold, (bigv, negv))
    carry = (cnt_p.astype(jnp.float32), cnt_v.astype(jnp.float32),
             pw, vw, cmin, cmax, rmin, rmax)

    for i in range(NACC):
        stage[i, :] = carry[i]
    pltpu.sync_copy(stage, shared.at[s])
    plsc.subcore_barrier()

    @pl.when(q == 0)
    def _combine():
        pltpu.sync_copy(shared.at[pl.ds(s, WPI)], gather)
        cnt_p = gather[0, 0, :] + gather[1, 0, :] + gather[2, 0, :] + gather[3, 0, :]
        cnt_v = gather[0, 1, :] + gather[1, 1, :] + gather[2, 1, :] + gather[3, 1, :]
        pw = gather[0, 2, :] + gather[1, 2, :] + gather[2, 2, :] + gather[3, 2, :]
        vw = gather[0, 3, :] + gather[1, 3, :] + gather[2, 3, :] + gather[3, 3, :]
        cmin = jnp.minimum(jnp.minimum(gather[0, 4, :], gather[1, 4, :]),
                           jnp.minimum(gather[2, 4, :], gather[3, 4, :]))
        cmax = jnp.maximum(jnp.maximum(gather[0, 5, :], gather[1, 5, :]),
                           jnp.maximum(gather[2, 5, :], gather[3, 5, :]))
        rmin = jnp.minimum(jnp.minimum(gather[0, 6, :], gather[1, 6, :]),
                           jnp.minimum(gather[2, 6, :], gather[3, 6, :]))
        rmax = jnp.maximum(jnp.maximum(gather[0, 7, :], gather[1, 7, :]),
                           jnp.maximum(gather[2, 7, :], gather[3, 7, :]))

        sp = jnp.max(cnt_p)
        sv = jnp.max(cnt_v)
        spw = jnp.sum(pw)
        svw = jnp.sum(vw)
        cmn = jnp.min(cmin)
        cmx = jnp.max(cmax)
        rmn = jnp.min(rmin)
        rmx = jnp.max(rmax)

        fg = sp + sv
        hp = sp > 0.0
        zero = jnp.float32(0.0)
        hr = jnp.where(hp, rmx - rmn, zero)
        wr = jnp.where(hp, cmx - cmn, zero)

        nums = (
            sp,
            sp,
            spw,
            hr,
            wr,
            2.0 * (hr + wr),
            svw,
            jnp.where(hp, spw, zero),
            fg,
            sp,
        )
        dens = (
            sv + 1e-6,
            fg + 1e-6,
            1.0,
            float(H),
            float(W),
            float(H + W),
            1.0,
            sp + 1e-6,
            float(H * W),
            float(H * W),
        )
        lane = lax.iota(jnp.int32, L)
        num = zv
        den = onev
        for k in range(10):
            sel = lane == k
            num = jnp.where(sel, nums[k], num)
            den = jnp.where(sel, dens[k], den)
        feat[:] = num / den
        pltpu.sync_copy(feat, out_hbm.at[b])


_mesh = plsc.VectorSubcoreMesh(
    core_axis_name="c", subcore_axis_name="s", num_cores=NC, num_subcores=NS
)

_sc_kernel = pl.kernel(
    _sc_body,
    out_type=jax.ShapeDtypeStruct((B, L), jnp.float32),
    mesh=_mesh,
    scratch_types=[
        pltpu.VMEM((2, C, RB, W), jnp.float32),
        pltpu.VMEM((NACC, L), jnp.float32),
        pltpu.VMEM((WPI, NACC, L), jnp.float32),
        pltpu.VMEM((L,), jnp.float32),
        pltpu.VMEM_SHARED((NS, NACC, L), jnp.float32),
        pltpu.SemaphoreType.DMA,
        pltpu.SemaphoreType.DMA,
    ],
    compiler_params=pltpu.CompilerParams(
        needs_layout_passes=False, skip_device_barrier=True
    ),
)


@jax.jit
def kernel(seg_logits):
    out = _sc_kernel(seg_logits)
    return out[:, :10]


# speedup vs baseline: 1.2424x; 1.0134x over previous
import jax
import jax.numpy as jnp
from jax import lax
from jax.experimental import pallas as pl
from jax.experimental.pallas import tpu as pltpu
from jax.experimental.pallas import tpu_sc as plsc

B, C, H, W = 8, 3, 512, 512
NC, NS, L = 2, 16, 16
IMGS_PER_CORE = B // NC
WPI = NS // IMGS_PER_CORE
ROWS_PER_WORKER = H // WPI
RB = 16
NCHUNK = ROWS_PER_WORKER // RB
NJ = W // L
NACC = 8


def _sc_body(seg_hbm, out_hbm, buf, stage, gather, feat, shared, sem0, sem1):
    c = lax.axis_index("c")
    s = lax.axis_index("s")
    img = s // WPI
    q = s % WPI
    b = c * IMGS_PER_CORE + img
    r0 = q * ROWS_PER_WORKER

    sems = (sem0, sem1)

    def start(g):
        return pltpu.async_copy(
            seg_hbm.at[b, :, pl.ds(r0 + g * RB, RB), :],
            buf.at[g % 2],
            sems[g % 2],
        )

    iota_f = lax.iota(jnp.int32, L).astype(jnp.float32)
    zv = jnp.zeros((L,), jnp.float32)
    onev = jnp.full((L,), 1.0, jnp.float32)
    bigv = jnp.full((L,), float(H + W), jnp.float32)
    negv = jnp.full((L,), -1.0, jnp.float32)

    zu = jnp.zeros((L,), jnp.uint32)
    oneu = jnp.full((L,), 1, jnp.uint32)
    zi = jnp.zeros((L,), jnp.int32)

    NR = 4

    carry = ((zi,) * NR, zi, zv, zv, zu, bigv, negv)

    desc = [None, None]
    desc[0] = start(0)
    for g in range(NCHUNK):
        if g + 1 < NCHUNK:
            desc[(g + 1) % 2] = start(g + 1)
        desc[g % 2].wait()

        def row_body(rr, car, g=g):
            cnt_ps, cnt_v, pw, vw, colbits, rmin, rmax = car
            prevs = cnt_ps
            rows = [NR * rr + k for k in range(NR)]

            def grp_body(j, gcar):
                cnt_ps, cnt_v, pw, vw, colbits, bb = gcar
                cnt_ps = list(cnt_ps)
                for k in range(NR):
                    r = rows[k]
                    l0 = buf[g % 2, 0, r, pl.ds(j * L, L)]
                    l1 = buf[g % 2, 1, r, pl.ds(j * L, L)]
                    l2 = buf[g % 2, 2, r, pl.ds(j * L, L)]
                    d1 = l1 - l0
                    d2 = l2 - l0
                    f1 = jnp.exp(d1)
                    f2 = jnp.exp(d2)
                    inv = 1.0 / (1.0 + f1 + f2)
                    fgm = jnp.maximum(l1, l2) > l0
                    c12 = l1 >= l2
                    plq = fgm & c12
                    ves = fgm ^ plq
                    cnt_ps[k] = cnt_ps[k] + plsc.all_reduce_population_count(plq)
                    cnt_v = cnt_v + plsc.all_reduce_population_count(ves)
                    pw = pw + jnp.where(plq, f1 * inv, zv)
                    vw = vw + jnp.where(ves, f2 * inv, zv)
                    colbits = colbits | jnp.where(plq, bb, zu)
                bb = bb << 1
                return (tuple(cnt_ps), cnt_v, pw, vw, colbits, bb)

            cnt_ps, cnt_v, pw, vw, colbits, _ = lax.fori_loop(
                0, NJ, grp_body, (cnt_ps, cnt_v, pw, vw, colbits, oneu)
            )
            for k in range(NR):
                rowf = (r0 + g * RB + rows[k]).astype(jnp.float32)
                rmask = cnt_ps[k] > prevs[k]
                rmin = jnp.minimum(rmin, jnp.where(rmask, rowf, bigv))
                rmax = jnp.maximum(rmax, jnp.where(rmask, rowf, negv))
            return (cnt_ps, cnt_v, pw, vw, colbits, rmin, rmax)

        carry = lax.fori_loop(0, RB // NR, row_body, carry)

    cnt_ps, cnt_v, pw, vw, colbits, rmin, rmax = carry
    cnt_p = cnt_ps[0] + cnt_ps[1] + cnt_ps[2] + cnt_ps[3]

    def col_fold(j, cc):
        cmin, cmax = cc
        anyv = ((colbits >> j) & oneu) > 0
        colf = iota_f + (j * L).astype(jnp.float32)
        cmin = jnp.minimum(cmin, jnp.where(anyv, colf, bigv))
        cmax = jnp.maximum(cmax, jnp.where(anyv, colf, negv))
        return (cmin, cmax)

    cmin, cmax = lax.fori_loop(0, NJ, col_fold, (bigv, negv))
    carry = (cnt_p.astype(jnp.float32), cnt_v.astype(jnp.float32),
             pw, vw, cmin, cmax, rmin, rmax)

    for i in range(NACC):
        stage[i, :] = carry[i]
    pltpu.sync_copy(stage, shared.at[s])
    plsc.subcore_barrier()

    @pl.when(q == 0)
    def _combine():
        pltpu.sync_copy(shared.at[pl.ds(s, WPI)], gather)
        cnt_p = gather[0, 0, :] + gather[1, 0, :] + gather[2, 0, :] + gather[3, 0, :]
        cnt_v = gather[0, 1, :] + gather[1, 1, :] + gather[2, 1, :] + gather[3, 1, :]
        pw = gather[0, 2, :] + gather[1, 2, :] + gather[2, 2, :] + gather[3, 2, :]
        vw = gather[0, 3, :] + gather[1, 3, :] + gather[2, 3, :] + gather[3, 3, :]
        cmin = jnp.minimum(jnp.minimum(gather[0, 4, :], gather[1, 4, :]),
                           jnp.minimum(gather[2, 4, :], gather[3, 4, :]))
        cmax = jnp.maximum(jnp.maximum(gather[0, 5, :], gather[1, 5, :]),
                           jnp.maximum(gather[2, 5, :], gather[3, 5, :]))
        rmin = jnp.minimum(jnp.minimum(gather[0, 6, :], gather[1, 6, :]),
                           jnp.minimum(gather[2, 6, :], gather[3, 6, :]))
        rmax = jnp.maximum(jnp.maximum(gather[0, 7, :], gather[1, 7, :]),
                           jnp.maximum(gather[2, 7, :], gather[3, 7, :]))

        sp = jnp.max(cnt_p)
        sv = jnp.max(cnt_v)
        spw = jnp.sum(pw)
        svw = jnp.sum(vw)
        cmn = jnp.min(cmin)
        cmx = jnp.max(cmax)
        rmn = jnp.min(rmin)
        rmx = jnp.max(rmax)

        fg = sp + sv
        hp = sp > 0.0
        zero = jnp.float32(0.0)
        hr = jnp.where(hp, rmx - rmn, zero)
        wr = jnp.where(hp, cmx - cmn, zero)

        nums = (
            sp,
            sp,
            spw,
            hr,
            wr,
            2.0 * (hr + wr),
            svw,
            jnp.where(hp, spw, zero),
            fg,
            sp,
        )
        dens = (
            sv + 1e-6,
            fg + 1e-6,
            1.0,
            float(H),
            float(W),
            float(H + W),
            1.0,
            sp + 1e-6,
            float(H * W),
            float(H * W),
        )
        lane = lax.iota(jnp.int32, L)
        num = zv
        den = onev
        for k in range(10):
            sel = lane == k
            num = jnp.where(sel, nums[k], num)
            den = jnp.where(sel, dens[k], den)
        feat[:] = num / den
        pltpu.sync_copy(feat, out_hbm.at[b])


_mesh = plsc.VectorSubcoreMesh(
    core_axis_name="c", subcore_axis_name="s", num_cores=NC, num_subcores=NS
)

_sc_kernel = pl.kernel(
    _sc_body,
    out_type=jax.ShapeDtypeStruct((B, L), jnp.float32),
    mesh=_mesh,
    scratch_types=[
        pltpu.VMEM((2, C, RB, W), jnp.float32),
        pltpu.VMEM((NACC, L), jnp.float32),
        pltpu.VMEM((WPI, NACC, L), jnp.float32),
        pltpu.VMEM((L,), jnp.float32),
        pltpu.VMEM_SHARED((NS, NACC, L), jnp.float32),
        pltpu.SemaphoreType.DMA,
        pltpu.SemaphoreType.DMA,
    ],
    compiler_params=pltpu.CompilerParams(needs_layout_passes=False),
)


@jax.jit
def kernel(seg_logits):
    out = _sc_kernel(seg_logits)
    return out[:, :10]


# speedup vs baseline: 1.2939x; 1.0414x over previous
import jax
import jax.numpy as jnp
from jax import lax
from jax.experimental import pallas as pl
from jax.experimental.pallas import tpu as pltpu
from jax.experimental.pallas import tpu_sc as plsc

B, C, H, W = 8, 3, 512, 512
NC, NS, L = 2, 16, 16
IMGS_PER_CORE = B // NC
WPI = NS // IMGS_PER_CORE
ROWS_PER_WORKER = H // WPI
RB = 16
NCHUNK = ROWS_PER_WORKER // RB
NJ = W // L
NACC = 8


def _sc_body(seg_hbm, out_hbm, buf, stage, gather, feat, shared, sem0, sem1):
    c = lax.axis_index("c")
    s = lax.axis_index("s")
    img = s // WPI
    q = s % WPI
    b = c * IMGS_PER_CORE + img
    r0 = q * ROWS_PER_WORKER

    sems = (sem0, sem1)

    def start(g):
        return pltpu.async_copy(
            seg_hbm.at[b, :, pl.ds(r0 + g * RB, RB), :],
            buf.at[g % 2],
            sems[g % 2],
        )

    iota_f = lax.iota(jnp.int32, L).astype(jnp.float32)
    zv = jnp.zeros((L,), jnp.float32)
    onev = jnp.full((L,), 1.0, jnp.float32)
    bigv = jnp.full((L,), float(H + W), jnp.float32)
    negv = jnp.full((L,), -1.0, jnp.float32)

    zu = jnp.zeros((L,), jnp.uint32)
    oneu = jnp.full((L,), 1, jnp.uint32)
    zi = jnp.zeros((L,), jnp.int32)

    NR = 4

    carry = ((zi,) * NR, zi, zv, zv, zu, bigv, negv)

    def process_chunk(par, g, car):
        def row_body(rr, car):
            cnt_ps, cnt_v, pw, vw, colbits, rmin, rmax = car
            prevs = cnt_ps

            def grp_body(j, gcar):
                cnt_ps, cnt_v, pw, vw, colbits, bb = gcar
                cnt_ps = list(cnt_ps)
                for k in range(NR):
                    r = NR * rr + k
                    l0 = buf[par, 0, r, pl.ds(j * L, L)]
                    l1 = buf[par, 1, r, pl.ds(j * L, L)]
                    l2 = buf[par, 2, r, pl.ds(j * L, L)]
                    d1 = l1 - l0
                    d2 = l2 - l0
                    f1 = jnp.exp(d1)
                    f2 = jnp.exp(d2)
                    inv = 1.0 / (1.0 + f1 + f2)
                    fgm = jnp.maximum(l1, l2) > l0
                    c12 = l1 >= l2
                    plq = fgm & c12
                    ves = fgm ^ plq
                    cnt_ps[k] = cnt_ps[k] + plsc.all_reduce_population_count(plq)
                    cnt_v = cnt_v + plsc.all_reduce_population_count(ves)
                    pw = pw + jnp.where(plq, f1 * inv, zv)
                    vw = vw + jnp.where(ves, f2 * inv, zv)
                    colbits = colbits | jnp.where(plq, bb, zu)
                bb = bb << 1
                return (tuple(cnt_ps), cnt_v, pw, vw, colbits, bb)

            cnt_ps, cnt_v, pw, vw, colbits, _ = lax.fori_loop(
                0, NJ, grp_body, (cnt_ps, cnt_v, pw, vw, colbits, oneu)
            )
            for k in range(NR):
                rowf = (r0 + g * RB + NR * rr + k).astype(jnp.float32)
                rmask = cnt_ps[k] > prevs[k]
                rmin = jnp.minimum(rmin, jnp.where(rmask, rowf, bigv))
                rmax = jnp.maximum(rmax, jnp.where(rmask, rowf, negv))
            return (cnt_ps, cnt_v, pw, vw, colbits, rmin, rmax)

        return lax.fori_loop(0, RB // NR, row_body, car)

    start(0)

    def chunk_pair(i, car):
        g0 = 2 * i
        pltpu.async_copy(
            seg_hbm.at[b, :, pl.ds(r0 + (g0 + 1) * RB, RB), :], buf.at[1], sem1
        )
        pltpu.make_async_copy(
            seg_hbm.at[b, :, pl.ds(r0 + g0 * RB, RB), :], buf.at[0], sem0
        ).wait()
        car = process_chunk(0, g0, car)

        @pl.when(i < NCHUNK // 2 - 1)
        def _():
            pltpu.async_copy(
                seg_hbm.at[b, :, pl.ds(r0 + (g0 + 2) * RB, RB), :],
                buf.at[0], sem0,
            )

        pltpu.make_async_copy(
            seg_hbm.at[b, :, pl.ds(r0 + (g0 + 1) * RB, RB), :], buf.at[1], sem1
        ).wait()
        car = process_chunk(1, g0 + 1, car)
        return car

    carry = lax.fori_loop(0, NCHUNK // 2, chunk_pair, carry)

    cnt_ps, cnt_v, pw, vw, colbits, rmin, rmax = carry
    cnt_p = cnt_ps[0] + cnt_ps[1] + cnt_ps[2] + cnt_ps[3]

    def col_fold(j, cc):
        cmin, cmax = cc
        anyv = ((colbits >> j) & oneu) > 0
        colf = iota_f + (j * L).astype(jnp.float32)
        cmin = jnp.minimum(cmin, jnp.where(anyv, colf, bigv))
        cmax = jnp.maximum(cmax, jnp.where(anyv, colf, negv))
        return (cmin, cmax)

    cmin, cmax = lax.fori_loop(0, NJ, col_fold, (bigv, negv))
    carry = (cnt_p.astype(jnp.float32), cnt_v.astype(jnp.float32),
             pw, vw, cmin, cmax, rmin, rmax)

    for i in range(NACC):
        stage[i, :] = carry[i]
    pltpu.sync_copy(stage, shared.at[s])
    plsc.subcore_barrier()

    @pl.when(q == 0)
    def _combine():
        pltpu.sync_copy(shared.at[pl.ds(s, WPI)], gather)
        cnt_p = gather[0, 0, :] + gather[1, 0, :] + gather[2, 0, :] + gather[3, 0, :]
        cnt_v = gather[0, 1, :] + gather[1, 1, :] + gather[2, 1, :] + gather[3, 1, :]
        pw = gather[0, 2, :] + gather[1, 2, :] + gather[2, 2, :] + gather[3, 2, :]
        vw = gather[0, 3, :] + gather[1, 3, :] + gather[2, 3, :] + gather[3, 3, :]
        cmin = jnp.minimum(jnp.minimum(gather[0, 4, :], gather[1, 4, :]),
                           jnp.minimum(gather[2, 4, :], gather[3, 4, :]))
        cmax = jnp.maximum(jnp.maximum(gather[0, 5, :], gather[1, 5, :]),
                           jnp.maximum(gather[2, 5, :], gather[3, 5, :]))
        rmin = jnp.minimum(jnp.minimum(gather[0, 6, :], gather[1, 6, :]),
                           jnp.minimum(gather[2, 6, :], gather[3, 6, :]))
        rmax = jnp.maximum(jnp.maximum(gather[0, 7, :], gather[1, 7, :]),
                           jnp.maximum(gather[2, 7, :], gather[3, 7, :]))

        sp = jnp.max(cnt_p)
        sv = jnp.max(cnt_v)
        spw = jnp.sum(pw)
        svw = jnp.sum(vw)
        cmn = jnp.min(cmin)
        cmx = jnp.max(cmax)
        rmn = jnp.min(rmin)
        rmx = jnp.max(rmax)

        fg = sp + sv
        hp = sp > 0.0
        zero = jnp.float32(0.0)
        hr = jnp.where(hp, rmx - rmn, zero)
        wr = jnp.where(hp, cmx - cmn, zero)

        nums = (
            sp,
            sp,
            spw,
            hr,
            wr,
            2.0 * (hr + wr),
            svw,
            jnp.where(hp, spw, zero),
            fg,
            sp,
        )
        dens = (
            sv + 1e-6,
            fg + 1e-6,
            1.0,
            float(H),
            float(W),
            float(H + W),
            1.0,
            sp + 1e-6,
            float(H * W),
            float(H * W),
        )
        lane = lax.iota(jnp.int32, L)
        num = zv
        den = onev
        for k in range(10):
            sel = lane == k
            num = jnp.where(sel, nums[k], num)
            den = jnp.where(sel, dens[k], den)
        feat[:] = num / den
        pltpu.sync_copy(feat, out_hbm.at[b])


_mesh = plsc.VectorSubcoreMesh(
    core_axis_name="c", subcore_axis_name="s", num_cores=NC, num_subcores=NS
)

_sc_kernel = pl.kernel(
    _sc_body,
    out_type=jax.ShapeDtypeStruct((B, L), jnp.float32),
    mesh=_mesh,
    scratch_types=[
        pltpu.VMEM((2, C, RB, W), jnp.float32),
        pltpu.VMEM((NACC, L), jnp.float32),
        pltpu.VMEM((WPI, NACC, L), jnp.float32),
        pltpu.VMEM((L,), jnp.float32),
        pltpu.VMEM_SHARED((NS, NACC, L), jnp.float32),
        pltpu.SemaphoreType.DMA,
        pltpu.SemaphoreType.DMA,
    ],
    compiler_params=pltpu.CompilerParams(needs_layout_passes=False),
)


@jax.jit
def kernel(seg_logits):
    out = _sc_kernel(seg_logits)
    return out[:, :10]


# speedup vs baseline: 1.2967x; 1.0022x over previous
import jax
import jax.numpy as jnp
from jax import lax
from jax.experimental import pallas as pl
from jax.experimental.pallas import tpu as pltpu
from jax.experimental.pallas import tpu_sc as plsc

B, C, H, W = 8, 3, 512, 512
NC, NS, L = 2, 16, 16
IMGS_PER_CORE = B // NC
WPI = NS // IMGS_PER_CORE
ROWS_PER_WORKER = H // WPI
RB = 16
NCHUNK = ROWS_PER_WORKER // RB
NJ = W // L
NACC = 8


def _sc_body(seg_hbm, out_hbm, buf, stage, gather, feat, shared, sem0, sem1):
    c = lax.axis_index("c")
    s = lax.axis_index("s")
    img = s // WPI
    q = s % WPI
    b = c * IMGS_PER_CORE + img
    r0 = q * ROWS_PER_WORKER

    sems = (sem0, sem1)

    def start(g):
        return pltpu.async_copy(
            seg_hbm.at[b, :, pl.ds(r0 + g * RB, RB), :],
            buf.at[g % 2],
            sems[g % 2],
        )

    iota_f = lax.iota(jnp.int32, L).astype(jnp.float32)
    zv = jnp.zeros((L,), jnp.float32)
    onev = jnp.full((L,), 1.0, jnp.float32)
    bigv = jnp.full((L,), float(H + W), jnp.float32)
    negv = jnp.full((L,), -1.0, jnp.float32)

    zu = jnp.zeros((L,), jnp.uint32)
    oneu = jnp.full((L,), 1, jnp.uint32)
    zi = jnp.zeros((L,), jnp.int32)

    NR = 4

    carry = ((zi,) * NR, zi, zv, zv, zu, bigv, negv)

    def process_chunk(par, g, car):
        def row_body(rr, car):
            cnt_ps, cnt_v, pw, vw, colbits, rmin, rmax = car
            prevs = cnt_ps

            def grp_body(j, gcar):
                cnt_ps, cnt_v, pw, vw, colbits, bb = gcar
                cnt_ps = list(cnt_ps)
                for k in range(NR):
                    r = NR * rr + k
                    l0 = buf[par, 0, r, pl.ds(j * L, L)]
                    l1 = buf[par, 1, r, pl.ds(j * L, L)]
                    l2 = buf[par, 2, r, pl.ds(j * L, L)]
                    d1 = l1 - l0
                    d2 = l2 - l0
                    f1 = jnp.exp(d1)
                    f2 = jnp.exp(d2)
                    inv = 1.0 / (1.0 + f1 + f2)
                    fgm = jnp.maximum(l1, l2) > l0
                    c12 = l1 >= l2
                    plq = fgm & c12
                    ves = fgm ^ plq
                    cnt_ps[k] = cnt_ps[k] + plsc.all_reduce_population_count(plq)
                    cnt_v = cnt_v + plsc.all_reduce_population_count(ves)
                    pw = pw + jnp.where(plq, f1 * inv, zv)
                    vw = vw + jnp.where(ves, f2 * inv, zv)
                    colbits = colbits | jnp.where(plq, bb, zu)
                bb = bb << 1
                return (tuple(cnt_ps), cnt_v, pw, vw, colbits, bb)

            cnt_ps, cnt_v, pw, vw, colbits, _ = plsc.parallel_loop(
                0, NJ, 1, unroll=1,
                carry=(cnt_ps, cnt_v, pw, vw, colbits, oneu)
            )(grp_body)
            for k in range(NR):
                rowf = (r0 + g * RB + NR * rr + k).astype(jnp.float32)
                rmask = cnt_ps[k] > prevs[k]
                rmin = jnp.minimum(rmin, jnp.where(rmask, rowf, bigv))
                rmax = jnp.maximum(rmax, jnp.where(rmask, rowf, negv))
            return (cnt_ps, cnt_v, pw, vw, colbits, rmin, rmax)

        return lax.fori_loop(0, RB // NR, row_body, car)

    start(0)

    def chunk_pair(i, car):
        g0 = 2 * i
        pltpu.async_copy(
            seg_hbm.at[b, :, pl.ds(r0 + (g0 + 1) * RB, RB), :], buf.at[1], sem1
        )
        pltpu.make_async_copy(
            seg_hbm.at[b, :, pl.ds(r0 + g0 * RB, RB), :], buf.at[0], sem0
        ).wait()
        car = process_chunk(0, g0, car)

        @pl.when(i < NCHUNK // 2 - 1)
        def _():
            pltpu.async_copy(
                seg_hbm.at[b, :, pl.ds(r0 + (g0 + 2) * RB, RB), :],
                buf.at[0], sem0,
            )

        pltpu.make_async_copy(
            seg_hbm.at[b, :, pl.ds(r0 + (g0 + 1) * RB, RB), :], buf.at[1], sem1
        ).wait()
        car = process_chunk(1, g0 + 1, car)
        return car

    carry = lax.fori_loop(0, NCHUNK // 2, chunk_pair, carry)

    cnt_ps, cnt_v, pw, vw, colbits, rmin, rmax = carry
    cnt_p = cnt_ps[0] + cnt_ps[1] + cnt_ps[2] + cnt_ps[3]

    def col_fold(j, cc):
        cmin, cmax = cc
        anyv = ((colbits >> j) & oneu) > 0
        colf = iota_f + (j * L).astype(jnp.float32)
        cmin = jnp.minimum(cmin, jnp.where(anyv, colf, bigv))
        cmax = jnp.maximum(cmax, jnp.where(anyv, colf, negv))
        return (cmin, cmax)

    cmin, cmax = lax.fori_loop(0, NJ, col_fold, (bigv, negv))
    carry = (cnt_p.astype(jnp.float32), cnt_v.astype(jnp.float32),
             pw, vw, cmin, cmax, rmin, rmax)

    for i in range(NACC):
        stage[i, :] = carry[i]
    pltpu.sync_copy(stage, shared.at[s])
    plsc.subcore_barrier()

    @pl.when(q == 0)
    def _combine():
        pltpu.sync_copy(shared.at[pl.ds(s, WPI)], gather)
        cnt_p = gather[0, 0, :] + gather[1, 0, :] + gather[2, 0, :] + gather[3, 0, :]
        cnt_v = gather[0, 1, :] + gather[1, 1, :] + gather[2, 1, :] + gather[3, 1, :]
        pw = gather[0, 2, :] + gather[1, 2, :] + gather[2, 2, :] + gather[3, 2, :]
        vw = gather[0, 3, :] + gather[1, 3, :] + gather[2, 3, :] + gather[3, 3, :]
        cmin = jnp.minimum(jnp.minimum(gather[0, 4, :], gather[1, 4, :]),
                           jnp.minimum(gather[2, 4, :], gather[3, 4, :]))
        cmax = jnp.maximum(jnp.maximum(gather[0, 5, :], gather[1, 5, :]),
                           jnp.maximum(gather[2, 5, :], gather[3, 5, :]))
        rmin = jnp.minimum(jnp.minimum(gather[0, 6, :], gather[1, 6, :]),
                           jnp.minimum(gather[2, 6, :], gather[3, 6, :]))
        rmax = jnp.maximum(jnp.maximum(gather[0, 7, :], gather[1, 7, :]),
                           jnp.maximum(gather[2, 7, :], gather[3, 7, :]))

        sp = jnp.max(cnt_p)
        sv = jnp.max(cnt_v)
        spw = jnp.sum(pw)
        svw = jnp.sum(vw)
        cmn = jnp.min(cmin)
        cmx = jnp.max(cmax)
        rmn = jnp.min(rmin)
        rmx = jnp.max(rmax)

        fg = sp + sv
        hp = sp > 0.0
        zero = jnp.float32(0.0)
        hr = jnp.where(hp, rmx - rmn, zero)
        wr = jnp.where(hp, cmx - cmn, zero)

        nums = (
            sp,
            sp,
            spw,
            hr,
            wr,
            2.0 * (hr + wr),
            svw,
            jnp.where(hp, spw, zero),
            fg,
            sp,
        )
        dens = (
            sv + 1e-6,
            fg + 1e-6,
            1.0,
            float(H),
            float(W),
            float(H + W),
            1.0,
            sp + 1e-6,
            float(H * W),
            float(H * W),
        )
        lane = lax.iota(jnp.int32, L)
        num = zv
        den = onev
        for k in range(10):
            sel = lane == k
            num = jnp.where(sel, nums[k], num)
            den = jnp.where(sel, dens[k], den)
        feat[:] = num / den
        pltpu.sync_copy(feat, out_hbm.at[b])


_mesh = plsc.VectorSubcoreMesh(
    core_axis_name="c", subcore_axis_name="s", num_cores=NC, num_subcores=NS
)

_sc_kernel = pl.kernel(
    _sc_body,
    out_type=jax.ShapeDtypeStruct((B, L), jnp.float32),
    mesh=_mesh,
    scratch_types=[
        pltpu.VMEM((2, C, RB, W), jnp.float32),
        pltpu.VMEM((NACC, L), jnp.float32),
        pltpu.VMEM((WPI, NACC, L), jnp.float32),
        pltpu.VMEM((L,), jnp.float32),
        pltpu.VMEM_SHARED((NS, NACC, L), jnp.float32),
        pltpu.SemaphoreType.DMA,
        pltpu.SemaphoreType.DMA,
    ],
    compiler_params=pltpu.CompilerParams(needs_layout_passes=False),
)


@jax.jit
def kernel(seg_logits):
    out = _sc_kernel(seg_logits)
    return out[:, :10]


# speedup vs baseline: 1.2982x; 1.0011x over previous
import jax
import jax.numpy as jnp
from jax import lax
from jax.experimental import pallas as pl
from jax.experimental.pallas import tpu as pltpu
from jax.experimental.pallas import tpu_sc as plsc

B, C, H, W = 8, 3, 512, 512
NC, NS, L = 2, 16, 16
IMGS_PER_CORE = B // NC
WPI = NS // IMGS_PER_CORE
ROWS_PER_WORKER = H // WPI
RB = 16
NCHUNK = ROWS_PER_WORKER // RB
NJ = W // L
NACC = 8


def _sc_body(seg_hbm, out_hbm, buf, stage, gather, feat, shared, sem0, sem1):
    c = lax.axis_index("c")
    s = lax.axis_index("s")
    img = s // WPI
    q = s % WPI
    b = c * IMGS_PER_CORE + img
    r0 = q * ROWS_PER_WORKER

    sems = (sem0, sem1)

    def start(g):
        return pltpu.async_copy(
            seg_hbm.at[b, :, pl.ds(r0 + g * RB, RB), :],
            buf.at[g % 2],
            sems[g % 2],
        )

    iota_f = lax.iota(jnp.int32, L).astype(jnp.float32)
    zv = jnp.zeros((L,), jnp.float32)
    onev = jnp.full((L,), 1.0, jnp.float32)
    bigv = jnp.full((L,), float(H + W), jnp.float32)
    negv = jnp.full((L,), -1.0, jnp.float32)

    zu = jnp.zeros((L,), jnp.uint32)
    oneu = jnp.full((L,), 1, jnp.uint32)
    zi = jnp.zeros((L,), jnp.int32)

    NR = 4

    carry = ((zi,) * NR, zi, zv, zv, zu, bigv, negv)

    def process_chunk(par, g, car):
        def row_body(rr, car):
            cnt_ps, cnt_v, pw, vw, colbits, rmin, rmax = car
            prevs = cnt_ps

            def grp_body(j, gcar):
                cnt_ps, cnt_v, pw, vw, colbits, bb = gcar
                cnt_ps = list(cnt_ps)
                for k in range(NR):
                    r = NR * rr + k
                    l0 = buf[par, 0, r, pl.ds(j * L, L)]
                    l1 = buf[par, 1, r, pl.ds(j * L, L)]
                    l2 = buf[par, 2, r, pl.ds(j * L, L)]
                    d1 = l1 - l0
                    d2 = l2 - l0
                    f1 = jnp.exp(d1)
                    f2 = jnp.exp(d2)
                    inv = 1.0 / (1.0 + f1 + f2)
                    fgm = jnp.maximum(l1, l2) > l0
                    c12 = l1 >= l2
                    plq = fgm & c12
                    ves = fgm ^ plq
                    cnt_ps[k] = cnt_ps[k] + plsc.all_reduce_population_count(plq)
                    cnt_v = cnt_v + plsc.all_reduce_population_count(ves)
                    pw = pw + jnp.where(plq, f1 * inv, zv)
                    vw = vw + jnp.where(ves, f2 * inv, zv)
                    colbits = colbits | jnp.where(plq, bb, zu)
                bb = bb << 1
                return (tuple(cnt_ps), cnt_v, pw, vw, colbits, bb)

            cnt_ps, cnt_v, pw, vw, colbits, _ = lax.fori_loop(
                0, NJ, grp_body, (cnt_ps, cnt_v, pw, vw, colbits, oneu)
            )
            for k in range(NR):
                rowf = (r0 + g * RB + NR * rr + k).astype(jnp.float32)
                rmask = cnt_ps[k] > prevs[k]
                rmin = jnp.minimum(rmin, jnp.where(rmask, rowf, bigv))
                rmax = jnp.maximum(rmax, jnp.where(rmask, rowf, negv))
            return (cnt_ps, cnt_v, pw, vw, colbits, rmin, rmax)

        return lax.fori_loop(0, RB // NR, row_body, car)

    start(0)

    def chunk_pair(i, car):
        g0 = 2 * i
        pltpu.async_copy(
            seg_hbm.at[b, :, pl.ds(r0 + (g0 + 1) * RB, RB), :], buf.at[1], sem1
        )
        pltpu.make_async_copy(
            seg_hbm.at[b, :, pl.ds(r0 + g0 * RB, RB), :], buf.at[0], sem0
        ).wait()
        car = process_chunk(0, g0, car)

        @pl.when(i < NCHUNK // 2 - 1)
        def _():
            pltpu.async_copy(
                seg_hbm.at[b, :, pl.ds(r0 + (g0 + 2) * RB, RB), :],
                buf.at[0], sem0,
            )

        pltpu.make_async_copy(
            seg_hbm.at[b, :, pl.ds(r0 + (g0 + 1) * RB, RB), :], buf.at[1], sem1
        ).wait()
        car = process_chunk(1, g0 + 1, car)
        return car

    carry = lax.fori_loop(0, NCHUNK // 2, chunk_pair, carry)

    cnt_ps, cnt_v, pw, vw, colbits, rmin, rmax = carry
    cnt_p = cnt_ps[0] + cnt_ps[1] + cnt_ps[2] + cnt_ps[3]

    def col_fold(j, cc):
        cmin, cmax = cc
        anyv = ((colbits >> j) & oneu) > 0
        colf = iota_f + (j * L).astype(jnp.float32)
        cmin = jnp.minimum(cmin, jnp.where(anyv, colf, bigv))
        cmax = jnp.maximum(cmax, jnp.where(anyv, colf, negv))
        return (cmin, cmax)

    cmin, cmax = lax.fori_loop(0, NJ, col_fold, (bigv, negv))
    carry = (cnt_p.astype(jnp.float32), cnt_v.astype(jnp.float32),
             pw, vw, cmin, cmax, rmin, rmax)

    for i in range(NACC):
        stage[i, :] = carry[i]
    pltpu.sync_copy(stage, shared.at[s])
    plsc.subcore_barrier()

    @pl.when(q == 0)
    def _combine():
        pltpu.sync_copy(shared.at[pl.ds(s, WPI)], gather)
        cnt_p = gather[0, 0, :] + gather[1, 0, :] + gather[2, 0, :] + gather[3, 0, :]
        cnt_v = gather[0, 1, :] + gather[1, 1, :] + gather[2, 1, :] + gather[3, 1, :]
        pw = gather[0, 2, :] + gather[1, 2, :] + gather[2, 2, :] + gather[3, 2, :]
        vw = gather[0, 3, :] + gather[1, 3, :] + gather[2, 3, :] + gather[3, 3, :]
        cmin = jnp.minimum(jnp.minimum(gather[0, 4, :], gather[1, 4, :]),
                           jnp.minimum(gather[2, 4, :], gather[3, 4, :]))
        cmax = jnp.maximum(jnp.maximum(gather[0, 5, :], gather[1, 5, :]),
                           jnp.maximum(gather[2, 5, :], gather[3, 5, :]))
        rmin = jnp.minimum(jnp.minimum(gather[0, 6, :], gather[1, 6, :]),
                           jnp.minimum(gather[2, 6, :], gather[3, 6, :]))
        rmax = jnp.maximum(jnp.maximum(gather[0, 7, :], gather[1, 7, :]),
                           jnp.maximum(gather[2, 7, :], gather[3, 7, :]))

        sp = jnp.max(cnt_p)
        sv = jnp.max(cnt_v)
        spw = jnp.sum(pw)
        svw = jnp.sum(vw)
        cmn = jnp.min(cmin)
        cmx = jnp.max(cmax)
        rmn = jnp.min(rmin)
        rmx = jnp.max(rmax)

        fg = sp + sv
        hp = sp > 0.0
        zero = jnp.float32(0.0)
        hr = jnp.where(hp, rmx - rmn, zero)
        wr = jnp.where(hp, cmx - cmn, zero)

        nums = (
            sp,
            sp,
            spw,
            hr,
            wr,
            2.0 * (hr + wr),
            svw,
            jnp.where(hp, spw, zero),
            fg,
            sp,
        )
        dens = (
            sv + 1e-6,
            fg + 1e-6,
            1.0,
            float(H),
            float(W),
            float(H + W),
            1.0,
            sp + 1e-6,
            float(H * W),
            float(H * W),
        )
        lane = lax.iota(jnp.int32, L)
        num = zv
        den = onev
        for k in range(10):
            sel = lane == k
            num = jnp.where(sel, nums[k], num)
            den = jnp.where(sel, dens[k], den)
        feat[:] = num / den
        pltpu.sync_copy(feat, out_hbm.at[b])


_mesh = plsc.VectorSubcoreMesh(
    core_axis_name="c", subcore_axis_name="s", num_cores=NC, num_subcores=NS
)

_sc_kernel = pl.kernel(
    _sc_body,
    out_type=jax.ShapeDtypeStruct((B, L), jnp.float32),
    mesh=_mesh,
    scratch_types=[
        pltpu.VMEM((2, C, RB, W), jnp.float32),
        pltpu.VMEM((NACC, L), jnp.float32),
        pltpu.VMEM((WPI, NACC, L), jnp.float32),
        pltpu.VMEM((L,), jnp.float32),
        pltpu.VMEM_SHARED((NS, NACC, L), jnp.float32),
        pltpu.SemaphoreType.DMA,
        pltpu.SemaphoreType.DMA,
    ],
    compiler_params=pltpu.CompilerParams(needs_layout_passes=False),
)


@jax.jit
def kernel(seg_logits):
    out = _sc_kernel(seg_logits)
    return out[:, :10]
